# baseline (speedup 1.0000x reference)
import functools

import jax
import jax.numpy as jnp
import numpy as np
from jax import lax
from jax.experimental import pallas as pl
from jax.experimental.pallas import tpu as pltpu

_NUM_HASHES = 5
_BITS = 12
_TABLE = 2048
_M = 3
_U = 0.83
_ROW_LEN = 64
_D = 512
_K = 4096
_QN = 8192
_N = _QN + _K
_HPAD = 8

_W_np = np.zeros((_NUM_HASHES * _BITS + 4, _HPAD), np.float32)
for _h in range(_NUM_HASHES):
    for _j in range(_BITS):
        _W_np[_h * _BITS + _j, _h] = float(2 ** _j) if _j < 11 else 0.0


def _codes_body(x_ref, at_ref, w_ref, out_ref):
    proj = jnp.dot(x_ref[...], at_ref[...], preferred_element_type=jnp.float32)
    bits = (proj > 0).astype(jnp.float32)
    out_ref[...] = jnp.dot(bits, w_ref[...], preferred_element_type=jnp.float32)


def _codes_call(x, at, w):
    tile = 1024
    grid = _N // tile
    return pl.pallas_call(
        _codes_body,
        grid=(grid,),
        in_specs=[
            pl.BlockSpec((tile, _D + _M), lambda i: (i, 0)),
            pl.BlockSpec((_D + _M, 64), lambda i: (0, 0)),
            pl.BlockSpec((64, _HPAD), lambda i: (0, 0)),
        ],
        out_specs=pl.BlockSpec((tile, _HPAD), lambda i: (i, 0)),
        out_shape=jax.ShapeDtypeStruct((_N, _HPAD), jnp.float32),
    )(x, at, w)


def _vote_body(qc_ref, kc3_ref, rows_ref):
    h = pl.program_id(0)
    qc = qc_ref[...]
    sel = (lax.broadcasted_iota(jnp.int32, (1, _HPAD), 1) == h).astype(jnp.float32)
    qcol = jnp.sum(qc * sel, axis=1, keepdims=True)
    hi = jnp.floor(qcol * (1.0 / 128.0))
    lo = qcol - hi * 128.0
    i16 = lax.broadcasted_iota(jnp.int32, (1, 16), 1).astype(jnp.float32)
    i128 = lax.broadcasted_iota(jnp.int32, (1, 128), 1).astype(jnp.float32)
    ehi = (hi == i16).astype(jnp.float32)
    elo = (lo == i128).astype(jnp.float32)
    counts = lax.dot_general(ehi, elo, (((0,), (0,)), ((), ())),
                             preferred_element_type=jnp.float32)
    maxv = jnp.max(counts)
    tids = (lax.broadcasted_iota(jnp.int32, (16, 128), 0) * 128
            + lax.broadcasted_iota(jnp.int32, (16, 128), 1)).astype(jnp.float32)
    best = jnp.min(jnp.where(counts == maxv, tids, 4096.0))
    kcv = kc3_ref[0]
    match = (kcv == best).astype(jnp.float32)
    tri = (lax.broadcasted_iota(jnp.int32, (128, 128), 0)
           <= lax.broadcasted_iota(jnp.int32, (128, 128), 1)).astype(jnp.float32)
    cumj = lax.dot_general(match, tri, (((1,), (0,)), ((), ())),
                           preferred_element_type=jnp.float32)
    ltri = (lax.broadcasted_iota(jnp.int32, (32, 32), 1)
            < lax.broadcasted_iota(jnp.int32, (32, 32), 0)).astype(jnp.float32)
    p = lax.dot_general(ltri, match, (((1,), (0,)), ((), ())),
                        preferred_element_type=jnp.float32)
    offs = jnp.sum(p, axis=1, keepdims=True)
    cum2 = cumj + offs
    iota64 = lax.broadcasted_iota(jnp.int32, (1, _ROW_LEN), 1)
    rows_f = jnp.zeros((1, _ROW_LEN), jnp.float32)
    for s in range(_ROW_LEN):
        cnt = jnp.sum((cum2 <= float(s)).astype(jnp.float32))
        rows_f = rows_f + cnt * (iota64 == s).astype(jnp.float32)
    rows_ref[0] = rows_f.astype(jnp.int32)


def _vote_call(qc, kc3):
    return pl.pallas_call(
        _vote_body,
        grid=(_NUM_HASHES,),
        in_specs=[
            pl.BlockSpec((_QN, _HPAD), lambda h: (0, 0)),
            pl.BlockSpec((1, 32, 128), lambda h: (h, 0, 0)),
        ],
        out_specs=pl.BlockSpec((1, 1, _ROW_LEN), lambda h: (h, 0, 0)),
        out_shape=jax.ShapeDtypeStruct((_NUM_HASHES, 1, _ROW_LEN), jnp.int32),
    )(qc, kc3)


def _score_body(rows_ref, q_ref, kz_ref, out_ref, active_ref):
    @pl.when(pl.program_id(0) == 0)
    def _():
        def body(s, carry):
            r = rows_ref[s]
            active_ref[pl.ds(s, 1), :] = kz_ref[pl.ds(r, 1), :]
            return carry
        lax.fori_loop(0, _NUM_HASHES * _ROW_LEN, body, 0)

    out_ref[...] = lax.dot_general(
        q_ref[...], active_ref[...], (((1,), (1,)), ((), ())),
        preferred_element_type=jnp.float32)


def _score_call(rows, queries, kz):
    tile = 512
    grid = _QN // tile
    nact = _NUM_HASHES * _ROW_LEN
    return pl.pallas_call(
        _score_body,
        grid=(grid,),
        in_specs=[
            pl.BlockSpec(memory_space=pltpu.SMEM),
            pl.BlockSpec((tile, _D), lambda i: (i, 0)),
            pl.BlockSpec((_K + 8, _D), lambda i: (0, 0)),
        ],
        out_specs=pl.BlockSpec((tile, nact), lambda i: (i, 0)),
        out_shape=jax.ShapeDtypeStruct((_QN, nact), jnp.float32),
        scratch_shapes=[pltpu.VMEM((nact, _D), jnp.float32)],
    )(rows, queries, kz)


def kernel(queries, kernels, a):
    norms = jnp.linalg.norm(kernels, axis=1)
    scale = _U / jnp.max(norms)
    ku = kernels * scale
    sq = jnp.sum(ku * ku, axis=1)
    aug_k = jnp.stack([sq ** (2 ** i) for i in range(_M)], axis=1)
    keys_aug = jnp.concatenate([ku, aug_k], axis=1)
    qn = queries / (jnp.linalg.norm(queries, axis=1, keepdims=True) + 1e-8)
    q_aug = jnp.concatenate(
        [qn, jnp.full((queries.shape[0], _M), 0.5, dtype=queries.dtype)], axis=1)
    x = jnp.concatenate([q_aug, keys_aug], axis=0)

    at = jnp.pad(a.T, ((0, 0), (0, 4)))
    w = jnp.asarray(_W_np[:64])
    codes = _codes_call(x, at, w)

    qc = codes[:_QN]
    kc3 = codes[_QN:, :_NUM_HASHES].T.reshape(_NUM_HASHES, 32, 128)
    rows = _vote_call(qc, kc3).reshape(_NUM_HASHES * _ROW_LEN)

    kz = jnp.concatenate([kernels, jnp.zeros((8, _D), kernels.dtype)], axis=0)
    return _score_call(rows, queries, kz)


# speedup vs baseline: 1.6982x; 1.6982x over previous
import functools

import jax
import jax.numpy as jnp
import numpy as np
from jax import lax
from jax.experimental import pallas as pl
from jax.experimental.pallas import tpu as pltpu

_NUM_HASHES = 5
_BITS = 12
_TABLE = 2048
_M = 3
_U = 0.83
_ROW_LEN = 64
_D = 512
_K = 4096
_QN = 8192
_N = _QN + _K
_HPAD = 8

_W_np = np.zeros((_NUM_HASHES * _BITS + 4, _HPAD), np.float32)
for _h in range(_NUM_HASHES):
    for _j in range(_BITS):
        _W_np[_h * _BITS + _j, _h] = float(2 ** _j) if _j < 11 else 0.0


_QT = _QN // 512
_KT = _K // 512


def _pack(x, at_ref, w_ref, out_ref):
    proj = jnp.dot(x, at_ref[...], preferred_element_type=jnp.float32)
    bits = (proj > 0).astype(jnp.float32)
    out_ref[...] = jnp.dot(bits, w_ref[...], preferred_element_type=jnp.float32)


def _codes_body(q_ref, k_ref, at_ref, w_ref, out_ref, maxsq_ref):
    i = pl.program_id(0)

    @pl.when(i < _KT)
    def _():
        k = k_ref[...]
        mx = jnp.max(jnp.sum(k * k, axis=1))

        @pl.when(i == 0)
        def _():
            maxsq_ref[0] = mx

        @pl.when(i > 0)
        def _():
            maxsq_ref[0] = jnp.maximum(maxsq_ref[0], mx)

    @pl.when((i >= _KT) & (i < 2 * _KT))
    def _():
        scale = _U / jnp.sqrt(maxsq_ref[0])
        ku = k_ref[...] * scale
        sq = jnp.sum(ku * ku, axis=1, keepdims=True)
        s2 = sq * sq
        s4 = s2 * s2
        x = jnp.concatenate([ku, sq, s2, s4], axis=1)
        _pack(x, at_ref, w_ref, out_ref)

    @pl.when(i >= 2 * _KT)
    def _():
        q = q_ref[...]
        nrm = jnp.sqrt(jnp.sum(q * q, axis=1, keepdims=True))
        qn = q / (nrm + 1e-8)
        half = jnp.full((q.shape[0], _M), 0.5, jnp.float32)
        x = jnp.concatenate([qn, half], axis=1)
        _pack(x, at_ref, w_ref, out_ref)


def _codes_call(queries, kernels, at, w):
    def qmap(i):
        return (lax.max(i - 2 * _KT, 0), 0)

    def kmap(i):
        return (jnp.clip(jnp.where(i < _KT, i, i - _KT), 0, _KT - 1), 0)

    def omap(i):
        return (jnp.where(i < _KT, _QT,
                jnp.where(i < 2 * _KT, _QT + (i - _KT), i - 2 * _KT)), 0)

    return pl.pallas_call(
        _codes_body,
        grid=(2 * _KT + _QT,),
        in_specs=[
            pl.BlockSpec((512, _D), qmap),
            pl.BlockSpec((512, _D), kmap),
            pl.BlockSpec((_D + _M, 64), lambda i: (0, 0)),
            pl.BlockSpec((64, _HPAD), lambda i: (0, 0)),
        ],
        out_specs=pl.BlockSpec((512, _HPAD), omap),
        out_shape=jax.ShapeDtypeStruct((_N, _HPAD), jnp.float32),
        scratch_shapes=[pltpu.SMEM((1,), jnp.float32)],
    )(queries, kernels, at, w)


def _vote_body(qc_ref, kc3_ref, rows_ref):
    h = pl.program_id(0)
    qc = qc_ref[...]
    sel = (lax.broadcasted_iota(jnp.int32, (1, _HPAD), 1) == h).astype(jnp.float32)
    qcol = jnp.sum(qc * sel, axis=1, keepdims=True)
    hi = jnp.floor(qcol * (1.0 / 128.0))
    lo = qcol - hi * 128.0
    i16 = lax.broadcasted_iota(jnp.int32, (1, 16), 1).astype(jnp.float32)
    i128 = lax.broadcasted_iota(jnp.int32, (1, 128), 1).astype(jnp.float32)
    ehi = (hi == i16).astype(jnp.float32)
    elo = (lo == i128).astype(jnp.float32)
    counts = lax.dot_general(ehi, elo, (((0,), (0,)), ((), ())),
                             preferred_element_type=jnp.float32)
    maxv = jnp.max(counts)
    tids = (lax.broadcasted_iota(jnp.int32, (16, 128), 0) * 128
            + lax.broadcasted_iota(jnp.int32, (16, 128), 1)).astype(jnp.float32)
    best = jnp.min(jnp.where(counts == maxv, tids, 4096.0))
    kcv = kc3_ref[0]
    match = (kcv == best).astype(jnp.float32)
    tri = (lax.broadcasted_iota(jnp.int32, (128, 128), 0)
           <= lax.broadcasted_iota(jnp.int32, (128, 128), 1)).astype(jnp.float32)
    cumj = lax.dot_general(match, tri, (((1,), (0,)), ((), ())),
                           preferred_element_type=jnp.float32)
    ltri = (lax.broadcasted_iota(jnp.int32, (32, 32), 1)
            < lax.broadcasted_iota(jnp.int32, (32, 32), 0)).astype(jnp.float32)
    p = lax.dot_general(ltri, match, (((1,), (0,)), ((), ())),
                        preferred_element_type=jnp.float32)
    offs = jnp.sum(p, axis=1, keepdims=True)
    cum2 = cumj + offs
    iota64 = lax.broadcasted_iota(jnp.int32, (1, _ROW_LEN), 1)
    rows_f = jnp.zeros((1, _ROW_LEN), jnp.float32)
    for s in range(_ROW_LEN):
        cnt = jnp.sum((cum2 <= float(s)).astype(jnp.float32))
        rows_f = rows_f + cnt * (iota64 == s).astype(jnp.float32)
    rows_ref[0] = rows_f.astype(jnp.int32)


def _vote_call(qc, kc3):
    return pl.pallas_call(
        _vote_body,
        grid=(_NUM_HASHES,),
        in_specs=[
            pl.BlockSpec((_QN, _HPAD), lambda h: (0, 0)),
            pl.BlockSpec((1, 32, 128), lambda h: (h, 0, 0)),
        ],
        out_specs=pl.BlockSpec((1, 1, _ROW_LEN), lambda h: (h, 0, 0)),
        out_shape=jax.ShapeDtypeStruct((_NUM_HASHES, 1, _ROW_LEN), jnp.int32),
    )(qc, kc3)


def _score_body(rows_ref, q_ref, k_ref, out_ref, active_ref):
    @pl.when(pl.program_id(0) == 0)
    def _():
        def body(s, carry):
            r = rows_ref[s]
            valid = jnp.where(r < _K, 1.0, 0.0)
            rc = jnp.minimum(r, _K - 1)
            active_ref[pl.ds(s, 1), :] = k_ref[pl.ds(rc, 1), :] * valid
            return carry
        lax.fori_loop(0, _NUM_HASHES * _ROW_LEN, body, 0)

    out_ref[...] = lax.dot_general(
        q_ref[...], active_ref[...], (((1,), (1,)), ((), ())),
        preferred_element_type=jnp.float32)


def _score_call(rows, queries, kernels):
    tile = 1024
    grid = _QN // tile
    nact = _NUM_HASHES * _ROW_LEN
    return pl.pallas_call(
        _score_body,
        grid=(grid,),
        in_specs=[
            pl.BlockSpec(memory_space=pltpu.SMEM),
            pl.BlockSpec((tile, _D), lambda i: (i, 0)),
            pl.BlockSpec((_K, _D), lambda i: (0, 0)),
        ],
        out_specs=pl.BlockSpec((tile, nact), lambda i: (i, 0)),
        out_shape=jax.ShapeDtypeStruct((_QN, nact), jnp.float32),
        scratch_shapes=[pltpu.VMEM((nact, _D), jnp.float32)],
    )(rows, queries, kernels)


def kernel(queries, kernels, a):
    at = jnp.pad(a.T, ((0, 0), (0, 4)))
    w = jnp.asarray(_W_np[:64])
    codes = _codes_call(queries, kernels, at, w)

    qc = codes[:_QN]
    kc3 = codes[_QN:, :_NUM_HASHES].T.reshape(_NUM_HASHES, 32, 128)
    rows = _vote_call(qc, kc3).reshape(_NUM_HASHES * _ROW_LEN)

    return _score_call(rows, queries, kernels)
